# Initial kernel scaffold; baseline (speedup 1.0000x reference)
import jax
import jax.numpy as jnp
from jax.experimental import pallas as pl


def kernel(weights, t):
    raise NotImplementedError("write your pallas kernel here")


# speedup vs baseline: 4.4078x; 4.4078x over previous
import functools
import jax
import jax.numpy as jnp
from jax import lax
from jax.experimental import pallas as pl
from jax.experimental.pallas import tpu as pltpu
from jax.experimental.pallas import tpu_sc as plsc

_B = 1024
_V = 200
_U1 = 101
_LANES = 16
_NCH = 7
_UP = _NCH * _LANES
_NW = 32
_BPW = _B // _NW


def _tec_kernel(w_hbm, g_hbm, t_hbm, ltab_hbm, sizes_hbm, seq_hbm,
                w_v, g_v, t_v, ltab_v, sizes_v, seq_v):
    cid = lax.axis_index("c")
    sid = lax.axis_index("s")
    wid = sid * 2 + cid
    base = wid * _BPW

    pltpu.sync_copy(t_hbm.at[pl.ds(base, _BPW)], t_v)
    pltpu.sync_copy(ltab_hbm, ltab_v)

    iota = lax.iota(jnp.int32, _LANES)
    lane0 = iota == 0
    zero_i = jnp.zeros((_LANES,), jnp.int32)
    zero_f = jnp.zeros((_LANES,), jnp.float32)
    neg_big = jnp.full((_LANES,), -1e30, jnp.float32)

    def b_body(bl, _):
        b = base + bl
        pltpu.sync_copy(w_hbm.at[b], w_v)
        pltpu.sync_copy(g_hbm.at[b], g_v)
        t_b = plsc.load_gather(t_v, [jnp.full((_LANES,), bl, jnp.int32)])

        def v_body(v, carry):
            size_vec = carry[0]
            matched = carry[1:]
            off = v * _UP
            w_ch = []
            g_ch = []
            cand = []
            k_acc = zero_i
            for i in range(_NCH):
                w_i = w_v[pl.ds(off + i * _LANES, _LANES)]
                g_i = g_v[pl.ds(off + i * _LANES, _LANES)]
                c_i = (w_i >= t_b) & (matched[i] == 0)
                w_ch.append(w_i)
                g_ch.append(g_i)
                cand.append(c_i)
                k_acc = k_acc + c_i.astype(jnp.int32)
            k = jnp.sum(k_acc)
            l_k = plsc.load_gather(ltab_v, [jnp.full((_LANES,), k, jnp.int32)])

            vals = [jnp.where(cand[i], g_ch[i] + l_k, neg_big)
                    for i in range(_NCH)]
            m01 = jnp.maximum(vals[0], vals[1])
            m23 = jnp.maximum(vals[2], vals[3])
            m45 = jnp.maximum(vals[4], vals[5])
            mx_vec = jnp.maximum(jnp.maximum(m01, m23),
                                 jnp.maximum(m45, vals[6]))
            mx = jnp.max(mx_vec)

            idxs = [jnp.where(vals[i] == mx, iota + i * _LANES, 10000)
                    for i in range(_NCH)]
            i01 = jnp.minimum(idxs[0], idxs[1])
            i23 = jnp.minimum(idxs[2], idxs[3])
            i45 = jnp.minimum(idxs[4], idxs[5])
            sel_vec = jnp.minimum(jnp.minimum(i01, i23),
                                  jnp.minimum(i45, idxs[6]))
            sel = jnp.min(sel_vec)

            w_sel = plsc.load_gather(
                w_v, [jnp.full((_LANES,), off + sel, jnp.int32)])
            size_vec = size_vec + w_sel

            take = sel > 0
            new_matched = tuple(
                jnp.where((iota + i * _LANES == sel) & take,
                          jnp.ones((_LANES,), jnp.int32), matched[i])
                for i in range(_NCH))
            plsc.store_scatter(seq_v, [jnp.full((_LANES,), v, jnp.int32)],
                               jnp.full((_LANES,), sel, jnp.int32),
                               mask=lane0)
            return (size_vec,) + new_matched

        init = (zero_f,) + tuple(zero_i for _ in range(_NCH))
        final = lax.fori_loop(0, _V, v_body, init)
        plsc.store_scatter(sizes_v, [jnp.full((_LANES,), bl, jnp.int32)],
                           final[0], mask=lane0)
        pltpu.sync_copy(seq_v, seq_hbm.at[b])
        return 0

    lax.fori_loop(0, _BPW, b_body, 0)
    pltpu.sync_copy(sizes_v, sizes_hbm.at[pl.ds(base, _BPW)])


@jax.jit
def _run(w2, g2, t1, ltab):
    mesh = plsc.VectorSubcoreMesh(core_axis_name="c", subcore_axis_name="s",
                                  num_cores=2, num_subcores=16)
    f = pl.kernel(
        _tec_kernel,
        out_type=[
            jax.ShapeDtypeStruct((_B,), jnp.float32),
            jax.ShapeDtypeStruct((_B, _V), jnp.int32),
        ],
        mesh=mesh,
        compiler_params=pltpu.CompilerParams(needs_layout_passes=False),
        scratch_types=[
            pltpu.VMEM((_V * _UP,), jnp.float32),
            pltpu.VMEM((_V * _UP,), jnp.float32),
            pltpu.VMEM((_BPW,), jnp.float32),
            pltpu.VMEM((104,), jnp.float32),
            pltpu.VMEM((_BPW,), jnp.float32),
            pltpu.VMEM((_V,), jnp.int32),
        ],
    )
    return f(w2, g2, t1, ltab)


def kernel(weights, t):
    B, V, U1 = weights.shape
    sample_key = jax.random.key(42)
    keys = jax.vmap(lambda v: jax.random.fold_in(sample_key, v))(jnp.arange(V))
    gum = jax.vmap(lambda k: jax.random.gumbel(k, (B, U1), jnp.float32))(keys)
    gum = jnp.transpose(gum, (1, 0, 2))
    w_pad = jnp.pad(weights, ((0, 0), (0, 0), (0, _UP - U1)))
    g_pad = jnp.pad(gum, ((0, 0), (0, 0), (0, _UP - U1)))
    w2 = w_pad.reshape(B, V * _UP)
    g2 = g_pad.reshape(B, V * _UP)
    counts = jnp.arange(104, dtype=jnp.float32).at[0].set(1.0)
    ltab = jnp.log(1.0 / counts)
    sizes, seq = _run(w2, g2, t.reshape(B), ltab)
    return (-sizes / V, seq.reshape(B, V, 1))


# speedup vs baseline: 5.3128x; 1.2053x over previous
import functools
import jax
import jax.numpy as jnp
from jax import lax
from jax.experimental import pallas as pl
from jax.experimental.pallas import tpu as pltpu
from jax.experimental.pallas import tpu_sc as plsc

_B = 1024
_V = 200
_U1 = 101
_LANES = 16
_NCH = 7
_UP = _NCH * _LANES
_NW = 32
_BPW = _B // _NW


def _tec_kernel(w_hbm, g_hbm, t_hbm, ltab_hbm, sizes_hbm, seq_hbm,
                w_a, g_a, w_b, g_b, t_v, ltab_v, sizes_v, seq_v,
                sem_wa, sem_ga, sem_wb, sem_gb):
    cid = lax.axis_index("c")
    sid = lax.axis_index("s")
    wid = sid * 2 + cid
    base = wid * _BPW

    pltpu.sync_copy(t_hbm.at[pl.ds(base, _BPW)], t_v)
    pltpu.sync_copy(ltab_hbm, ltab_v)

    iota = lax.iota(jnp.int32, _LANES)
    lane0 = iota == 0
    perms = [(iota ^ s)[:, None] for s in (8, 4, 2, 1)]
    dnums = lax.GatherDimensionNumbers(
        offset_dims=(), collapsed_slice_dims=(0,), start_index_map=(0,))

    def shuf(x, p):
        return lax.gather(x, p, dnums, slice_sizes=(1,),
                          mode=lax.GatherScatterMode.PROMISE_IN_BOUNDS)
    zero_i = jnp.zeros((_LANES,), jnp.int32)
    zero_f = jnp.zeros((_LANES,), jnp.float32)
    one_i = jnp.ones((_LANES,), jnp.int32)
    neg_big = jnp.full((_LANES,), -1e30, jnp.float32)

    def start(b, w_buf, g_buf, sem_w, sem_g):
        pltpu.make_async_copy(w_hbm.at[b], w_buf, sem_w).start()
        pltpu.make_async_copy(g_hbm.at[b], g_buf, sem_g).start()

    def wait(b, w_buf, g_buf, sem_w, sem_g):
        pltpu.make_async_copy(w_hbm.at[b], w_buf, sem_w).wait()
        pltpu.make_async_copy(g_hbm.at[b], g_buf, sem_g).wait()

    def process(bl, w_v, g_v):
        t_b = plsc.load_gather(t_v, [jnp.full((_LANES,), bl, jnp.int32)])

        def v_body(v, carry):
            size_vec = carry[0]
            matched = carry[1:]
            off = v * _UP
            g_ch = []
            cand = []
            gc = []
            for i in range(_NCH):
                w_i = w_v[pl.ds(off + i * _LANES, _LANES)]
                g_i = g_v[pl.ds(off + i * _LANES, _LANES)]
                c_i = (w_i >= t_b) & (matched[i] == 0)
                g_ch.append(g_i)
                cand.append(c_i)
                gc.append(jnp.where(c_i, g_i, neg_big))

            kv = plsc.all_reduce_population_count(cand[0])
            for i in range(1, _NCH):
                kv = kv + plsc.all_reduce_population_count(cand[i])
            l_k = plsc.load_gather(ltab_v, [kv])

            m01 = jnp.maximum(gc[0], gc[1])
            m23 = jnp.maximum(gc[2], gc[3])
            m45 = jnp.maximum(gc[4], gc[5])
            m = jnp.maximum(jnp.maximum(m01, m23), jnp.maximum(m45, gc[6]))
            for p in perms:
                m = jnp.maximum(m, shuf(m, p))
            big_m = m + l_k

            vals = [jnp.where(cand[i], g_ch[i] + l_k, neg_big)
                    for i in range(_NCH)]
            idxs = [jnp.where(vals[i] == big_m, iota + i * _LANES,
                              jnp.full((_LANES,), 10000, jnp.int32))
                    for i in range(_NCH)]
            i01 = jnp.minimum(idxs[0], idxs[1])
            i23 = jnp.minimum(idxs[2], idxs[3])
            i45 = jnp.minimum(idxs[4], idxs[5])
            sel = jnp.minimum(jnp.minimum(i01, i23),
                              jnp.minimum(i45, idxs[6]))
            for p in perms:
                sel = jnp.minimum(sel, shuf(sel, p))

            w_sel = plsc.load_gather(w_v, [off + sel])
            size_vec = size_vec + w_sel

            take = sel > 0
            new_matched = tuple(
                jnp.where((iota + i * _LANES == sel) & take, one_i, matched[i])
                for i in range(_NCH))
            plsc.store_scatter(seq_v, [jnp.full((_LANES,), v, jnp.int32)],
                               sel, mask=lane0)
            return (size_vec,) + new_matched

        init = (zero_f,) + tuple(zero_i for _ in range(_NCH))
        final = lax.fori_loop(0, _V, v_body, init)
        plsc.store_scatter(sizes_v, [jnp.full((_LANES,), bl, jnp.int32)],
                           final[0], mask=lane0)

    start(base, w_a, g_a, sem_wa, sem_ga)

    def b_body(j, _):
        b0 = base + 2 * j
        b1 = b0 + 1
        b2 = jnp.minimum(b0 + 2, _B - 1)
        wait(b0, w_a, g_a, sem_wa, sem_ga)
        start(b1, w_b, g_b, sem_wb, sem_gb)
        process(2 * j, w_a, g_a)
        pltpu.sync_copy(seq_v, seq_hbm.at[b0])
        wait(b1, w_b, g_b, sem_wb, sem_gb)
        start(b2, w_a, g_a, sem_wa, sem_ga)
        process(2 * j + 1, w_b, g_b)
        pltpu.sync_copy(seq_v, seq_hbm.at[b1])
        return 0

    lax.fori_loop(0, _BPW // 2, b_body, 0)
    wait(_B - 1, w_a, g_a, sem_wa, sem_ga)
    pltpu.sync_copy(sizes_v, sizes_hbm.at[pl.ds(base, _BPW)])


@jax.jit
def _run(w2, g2, t1, ltab):
    mesh = plsc.VectorSubcoreMesh(core_axis_name="c", subcore_axis_name="s",
                                  num_cores=2, num_subcores=16)
    f = pl.kernel(
        _tec_kernel,
        out_type=[
            jax.ShapeDtypeStruct((_B,), jnp.float32),
            jax.ShapeDtypeStruct((_B, _V), jnp.int32),
        ],
        mesh=mesh,
        compiler_params=pltpu.CompilerParams(needs_layout_passes=False),
        scratch_types=[
            pltpu.VMEM((_V * _UP,), jnp.float32),
            pltpu.VMEM((_V * _UP,), jnp.float32),
            pltpu.VMEM((_V * _UP,), jnp.float32),
            pltpu.VMEM((_V * _UP,), jnp.float32),
            pltpu.VMEM((_BPW,), jnp.float32),
            pltpu.VMEM((104,), jnp.float32),
            pltpu.VMEM((_BPW,), jnp.float32),
            pltpu.VMEM((_V,), jnp.int32),
            pltpu.SemaphoreType.DMA,
            pltpu.SemaphoreType.DMA,
            pltpu.SemaphoreType.DMA,
            pltpu.SemaphoreType.DMA,
        ],
    )
    return f(w2, g2, t1, ltab)


def kernel(weights, t):
    B, V, U1 = weights.shape
    sample_key = jax.random.key(42)
    keys = jax.vmap(lambda v: jax.random.fold_in(sample_key, v))(jnp.arange(V))
    gum = jax.vmap(lambda k: jax.random.gumbel(k, (B, U1), jnp.float32))(keys)
    gum = jnp.transpose(gum, (1, 0, 2))
    w_pad = jnp.pad(weights, ((0, 0), (0, 0), (0, _UP - U1)))
    g_pad = jnp.pad(gum, ((0, 0), (0, 0), (0, _UP - U1)))
    w2 = w_pad.reshape(B, V * _UP)
    g2 = g_pad.reshape(B, V * _UP)
    counts = jnp.arange(104, dtype=jnp.float32).at[0].set(1.0)
    ltab = jnp.log(1.0 / counts)
    sizes, seq = _run(w2, g2, t.reshape(B), ltab)
    return (-sizes / V, seq.reshape(B, V, 1))


# speedup vs baseline: 8.8550x; 1.6667x over previous
import functools
import jax
import jax.numpy as jnp
from jax import lax
from jax.experimental import pallas as pl
from jax.experimental.pallas import tpu as pltpu
from jax.experimental.pallas import tpu_sc as plsc

_B = 1024
_V = 200
_U1 = 101
_LANES = 16
_NCH = 7
_UP = _NCH * _LANES
_NW = 32
_BPW = _B // _NW


def _tec_kernel(w_hbm, g_hbm, t_hbm, ltab_hbm, sizes_hbm, seq_hbm,
                w_a, g_a, w_b, g_b, t_v, ltab_v, sizes_v, seq_v,
                sem_wa, sem_ga, sem_wb, sem_gb):
    cid = lax.axis_index("c")
    sid = lax.axis_index("s")
    wid = sid * 2 + cid
    base = wid * _BPW

    pltpu.sync_copy(t_hbm.at[pl.ds(base, _BPW)], t_v)
    pltpu.sync_copy(ltab_hbm, ltab_v)

    iota = lax.iota(jnp.int32, _LANES)
    lane0 = iota == 0
    perms = [(iota ^ s)[:, None] for s in (8, 4, 2, 1)]
    dnums = lax.GatherDimensionNumbers(
        offset_dims=(), collapsed_slice_dims=(0,), start_index_map=(0,))

    def shuf(x, p):
        return lax.gather(x, p, dnums, slice_sizes=(1,),
                          mode=lax.GatherScatterMode.PROMISE_IN_BOUNDS)
    zero_i = jnp.zeros((_LANES,), jnp.int32)
    zero_f = jnp.zeros((_LANES,), jnp.float32)
    one_i = jnp.ones((_LANES,), jnp.int32)
    neg_big = jnp.full((_LANES,), -1e30, jnp.float32)

    def start(b, w_buf, g_buf, sem_w, sem_g):
        pltpu.make_async_copy(w_hbm.at[b], w_buf, sem_w).start()
        pltpu.make_async_copy(g_hbm.at[b], g_buf, sem_g).start()

    def wait(b, w_buf, g_buf, sem_w, sem_g):
        pltpu.make_async_copy(w_hbm.at[b], w_buf, sem_w).wait()
        pltpu.make_async_copy(g_hbm.at[b], g_buf, sem_g).wait()

    def process(bl, w_v, g_v):
        t_b = plsc.load_gather(t_v, [jnp.full((_LANES,), bl, jnp.int32)])

        def v_body(v, carry):
            size_vec = carry[0]
            matched = carry[1:]
            off = v * _UP
            g_ch = []
            cand = []
            gc = []
            for i in range(_NCH):
                w_i = w_v[pl.ds(off + i * _LANES, _LANES)]
                g_i = g_v[pl.ds(off + i * _LANES, _LANES)]
                c_i = (w_i >= t_b) & (matched[i] == 0)
                g_ch.append(g_i)
                cand.append(c_i)
                gc.append(jnp.where(c_i, g_i, neg_big))

            kv = plsc.all_reduce_population_count(cand[0])
            for i in range(1, _NCH):
                kv = kv + plsc.all_reduce_population_count(cand[i])
            l_k = plsc.load_gather(ltab_v, [kv])

            m01 = jnp.maximum(gc[0], gc[1])
            m23 = jnp.maximum(gc[2], gc[3])
            m45 = jnp.maximum(gc[4], gc[5])
            m = jnp.maximum(jnp.maximum(m01, m23), jnp.maximum(m45, gc[6]))
            for p in perms:
                m = jnp.maximum(m, shuf(m, p))
            big_m = m + l_k

            vals = [jnp.where(cand[i], g_ch[i] + l_k, neg_big)
                    for i in range(_NCH)]
            idxs = [jnp.where(vals[i] == big_m, iota + i * _LANES,
                              jnp.full((_LANES,), 10000, jnp.int32))
                    for i in range(_NCH)]
            i01 = jnp.minimum(idxs[0], idxs[1])
            i23 = jnp.minimum(idxs[2], idxs[3])
            i45 = jnp.minimum(idxs[4], idxs[5])
            sel = jnp.minimum(jnp.minimum(i01, i23),
                              jnp.minimum(i45, idxs[6]))
            for p in perms:
                sel = jnp.minimum(sel, shuf(sel, p))

            w_sel = plsc.load_gather(w_v, [off + sel])
            size_vec = size_vec + w_sel

            take = sel > 0
            new_matched = tuple(
                jnp.where((iota + i * _LANES == sel) & take, one_i, matched[i])
                for i in range(_NCH))
            plsc.store_scatter(seq_v, [jnp.full((_LANES,), v, jnp.int32)],
                               sel, mask=lane0)
            return (size_vec,) + new_matched

        init = (zero_f,) + tuple(zero_i for _ in range(_NCH))
        final = lax.fori_loop(0, _V, v_body, init)
        plsc.store_scatter(sizes_v, [jnp.full((_LANES,), bl, jnp.int32)],
                           final[0], mask=lane0)

    start(base, w_a, g_a, sem_wa, sem_ga)

    def b_body(j, _):
        b0 = base + 2 * j
        b1 = b0 + 1
        b2 = jnp.minimum(b0 + 2, _B - 1)
        wait(b0, w_a, g_a, sem_wa, sem_ga)
        start(b1, w_b, g_b, sem_wb, sem_gb)
        process(2 * j, w_a, g_a)
        pltpu.sync_copy(seq_v, seq_hbm.at[b0])
        wait(b1, w_b, g_b, sem_wb, sem_gb)
        start(b2, w_a, g_a, sem_wa, sem_ga)
        process(2 * j + 1, w_b, g_b)
        pltpu.sync_copy(seq_v, seq_hbm.at[b1])
        return 0

    lax.fori_loop(0, _BPW // 2, b_body, 0)
    wait(_B - 1, w_a, g_a, sem_wa, sem_ga)
    pltpu.sync_copy(sizes_v, sizes_hbm.at[pl.ds(base, _BPW)])


@jax.jit
def _run(w2, g2, t1, ltab):
    mesh = plsc.VectorSubcoreMesh(core_axis_name="c", subcore_axis_name="s",
                                  num_cores=2, num_subcores=16)
    f = pl.kernel(
        _tec_kernel,
        out_type=[
            jax.ShapeDtypeStruct((_B,), jnp.float32),
            jax.ShapeDtypeStruct((_B, _V), jnp.int32),
        ],
        mesh=mesh,
        compiler_params=pltpu.CompilerParams(needs_layout_passes=False),
        scratch_types=[
            pltpu.VMEM((_V * _UP,), jnp.float32),
            pltpu.VMEM((_V * _UP,), jnp.float32),
            pltpu.VMEM((_V * _UP,), jnp.float32),
            pltpu.VMEM((_V * _UP,), jnp.float32),
            pltpu.VMEM((_BPW,), jnp.float32),
            pltpu.VMEM((104,), jnp.float32),
            pltpu.VMEM((_BPW,), jnp.float32),
            pltpu.VMEM((_V,), jnp.int32),
            pltpu.SemaphoreType.DMA,
            pltpu.SemaphoreType.DMA,
            pltpu.SemaphoreType.DMA,
            pltpu.SemaphoreType.DMA,
        ],
    )
    return f(w2, g2, t1, ltab)


def kernel(weights, t):
    B, V, U1 = weights.shape
    with jax.ensure_compile_time_eval():
        sample_key = jax.random.key(42)
        keys = jax.vmap(
            lambda v: jax.random.fold_in(sample_key, v))(jnp.arange(V))
        gum = jax.vmap(
            lambda k: jax.random.gumbel(k, (B, U1), jnp.float32))(keys)
        gum = jnp.transpose(gum, (1, 0, 2))
        g_pad = jnp.pad(gum, ((0, 0), (0, 0), (0, _UP - U1)))
        g2 = g_pad.reshape(B, V * _UP)
        counts = jnp.arange(104, dtype=jnp.float32).at[0].set(1.0)
        ltab = jnp.log(1.0 / counts)
    w_pad = jnp.pad(weights, ((0, 0), (0, 0), (0, _UP - U1)))
    w2 = w_pad.reshape(B, V * _UP)
    sizes, seq = _run(w2, g2, t.reshape(B), ltab)
    return (-sizes / V, seq.reshape(B, V, 1))
